# baseline (speedup 1.0000x reference)
import jax
import jax.numpy as jnp
from jax.experimental import pallas as pl
from jax.experimental.pallas import tpu as pltpu

_BN = 32768
_RB = _BN // 128


def _fused_kernel(xa_ref, xb_ref, xc_ref, xd_ref, w_ref, b_ref, o_ref, s_scr, r_scr, sm):
    p = pl.program_id(0)
    j = pl.program_id(1)

    @pl.when(p == 0)
    def _produce():
        @pl.when(j == 0)
        def _init():
            sm[0] = 0
        wsum = jnp.sum(w_ref[...], axis=0, keepdims=True)
        bsum = jnp.sum(b_ref[...])
        cols = [jax.lax.dot_general(
            xr[...], wsum,
            dimension_numbers=(((1,), (1,)), ((), ())),
            preferred_element_type=jnp.float32)
            for xr in (xa_ref, xb_ref, xc_ref, xd_ref)]
        s = jnp.concatenate(
            [c.reshape(_RB // 4, 128) for c in cols], axis=0) + bsum
        nz = (s != 0.0)
        mi = nz.astype(jnp.float32)
        d = jax.lax.broadcasted_iota(jnp.int32, (128, 128), 0)
        l = jax.lax.broadcasted_iota(jnp.int32, (128, 128), 1)
        tri = (d <= l).astype(jnp.float32)
        incl = jax.lax.dot(mi, tri,
                           preferred_element_type=jnp.float32)
        sel = (d == 127).astype(jnp.float32)
        rowcnt = jax.lax.dot(incl, sel,
                             preferred_element_type=jnp.float32)
        r2 = jax.lax.broadcasted_iota(jnp.int32, (_RB, _RB), 0)
        q2 = jax.lax.broadcasted_iota(jnp.int32, (_RB, _RB), 1)
        low = (q2 < r2).astype(jnp.float32)
        rowoff = jax.lax.dot(low, rowcnt,
                             preferred_element_type=jnp.float32)
        carry = sm[0].astype(jnp.float32)
        rank = carry + rowoff + (incl - mi)
        s_scr[pl.ds(j * _RB, _RB), :] = s
        r_scr[pl.ds(j * _RB, _RB), :] = rank
        sm[0] = sm[0] + jnp.sum(mi).astype(jnp.int32)

    @pl.when(p == 1)
    def _emit():
        k = sm[0]
        start = jnp.maximum(k // 2, 1).astype(jnp.float32)
        s = s_scr[pl.ds(j * _RB, _RB), :]
        rank = r_scr[pl.ds(j * _RB, _RB), :]
        keep = (s != 0.0) & (rank >= start)
        o_ref[...] = jnp.where(keep, s, 0.0)


def kernel(x, W, b):
    N, D = x.shape
    R = N // 128
    NB = N // _BN
    b2d = b.reshape(1, b.shape[0])
    out2d = pl.pallas_call(
        _fused_kernel,
        grid=(2, NB),
        in_specs=[
            pl.BlockSpec((_BN // 4, D),
                         lambda p, j, q=q: (4 * (j * (1 - p) + (NB - 1) * p) + q, 0))
            for q in range(4)] + [
            pl.BlockSpec((W.shape[0], D), lambda p, j: (0, 0)),
            pl.BlockSpec((1, b.shape[0]), lambda p, j: (0, 0)),
        ],
        out_specs=pl.BlockSpec((_RB, 128), lambda p, j: (j * p, 0)),
        out_shape=jax.ShapeDtypeStruct((R, 128), jnp.float32),
        scratch_shapes=[
            pltpu.VMEM((R, 128), jnp.float32),
            pltpu.VMEM((R, 128), jnp.float32),
            pltpu.SMEM((1,), jnp.int32),
        ],
        compiler_params=pltpu.CompilerParams(
            dimension_semantics=("arbitrary", "arbitrary")),
    )(x, x, x, x, W, b2d)
    return out2d.reshape(N)


# speedup vs baseline: 1.0148x; 1.0148x over previous
import jax
import jax.numpy as jnp
from jax.experimental import pallas as pl
from jax.experimental.pallas import tpu as pltpu

_BN = 32768
_RB = _BN // 128


def _fused_kernel(xa_ref, xb_ref, w_ref, b_ref, o_ref, s_scr, r_scr, sm):
    p = pl.program_id(0)
    j = pl.program_id(1)

    @pl.when(p == 0)
    def _produce():
        @pl.when(j == 0)
        def _init():
            sm[0] = 0
        wsum = jnp.sum(w_ref[...], axis=0, keepdims=True)
        bsum = jnp.sum(b_ref[...])
        cola = jax.lax.dot_general(
            xa_ref[...], wsum,
            dimension_numbers=(((1,), (1,)), ((), ())),
            preferred_element_type=jnp.float32)
        colb = jax.lax.dot_general(
            xb_ref[...], wsum,
            dimension_numbers=(((1,), (1,)), ((), ())),
            preferred_element_type=jnp.float32)
        s = jnp.concatenate(
            [cola.reshape(_RB // 2, 128), colb.reshape(_RB // 2, 128)],
            axis=0) + bsum
        nz = (s != 0.0)
        mi = nz.astype(jnp.float32)
        d = jax.lax.broadcasted_iota(jnp.int32, (128, 128), 0)
        l = jax.lax.broadcasted_iota(jnp.int32, (128, 128), 1)
        tri = (d <= l).astype(jnp.float32)
        incl = jax.lax.dot(mi, tri,
                           preferred_element_type=jnp.float32)
        sel = (d == 127).astype(jnp.float32)
        rowcnt = jax.lax.dot(incl, sel,
                             preferred_element_type=jnp.float32)
        r2 = jax.lax.broadcasted_iota(jnp.int32, (_RB, _RB), 0)
        q2 = jax.lax.broadcasted_iota(jnp.int32, (_RB, _RB), 1)
        low = (q2 < r2).astype(jnp.float32)
        rowoff = jax.lax.dot(low, rowcnt,
                             preferred_element_type=jnp.float32)
        carry = sm[0].astype(jnp.float32)
        rank = carry + rowoff + (incl - mi)
        s_scr[pl.ds(j * _RB, _RB), :] = s
        r_scr[pl.ds(j * _RB, _RB), :] = rank
        sm[0] = sm[0] + jnp.sum(mi).astype(jnp.int32)

    @pl.when(p == 1)
    def _emit():
        k = sm[0]
        start = jnp.maximum(k // 2, 1).astype(jnp.float32)
        s = s_scr[pl.ds(j * _RB, _RB), :]
        rank = r_scr[pl.ds(j * _RB, _RB), :]
        keep = (s != 0.0) & (rank >= start)
        o_ref[...] = jnp.where(keep, s, 0.0)


def kernel(x, W, b):
    N, D = x.shape
    R = N // 128
    NB = N // _BN
    b2d = b.reshape(1, b.shape[0])
    out2d = pl.pallas_call(
        _fused_kernel,
        grid=(2, NB),
        in_specs=[
            pl.BlockSpec((_BN // 2, D),
                         lambda p, j: (2 * (j * (1 - p) + (NB - 1) * p), 0)),
            pl.BlockSpec((_BN // 2, D),
                         lambda p, j: (2 * (j * (1 - p) + (NB - 1) * p) + 1, 0)),
            pl.BlockSpec((W.shape[0], D), lambda p, j: (0, 0)),
            pl.BlockSpec((1, b.shape[0]), lambda p, j: (0, 0)),
        ],
        out_specs=pl.BlockSpec((_RB, 128), lambda p, j: (j * p, 0)),
        out_shape=jax.ShapeDtypeStruct((R, 128), jnp.float32),
        scratch_shapes=[
            pltpu.VMEM((R, 128), jnp.float32),
            pltpu.VMEM((R, 128), jnp.float32),
            pltpu.SMEM((1,), jnp.int32),
        ],
        compiler_params=pltpu.CompilerParams(
            dimension_semantics=("arbitrary", "arbitrary")),
    )(x, x, W, b2d)
    return out2d.reshape(N)
